# baseline (speedup 1.0000x reference)
import functools

import jax
import jax.numpy as jnp
from jax import lax
from jax.experimental import pallas as pl
from jax.experimental.pallas import tpu as pltpu
from jax.experimental.pallas import tpu_sc as plsc

NC, NS = 2, 16
NW = NC * NS
EPS = 1e-8
BN_EPS = 1e-5


def _row_scores(table, w, rows_per_block):
    n, e = table.shape
    s = 125
    r = n // s
    t3 = table.reshape(r, s, e)

    def body(t_ref, w_ref, o_ref):
        o_ref[...] = lax.dot_general(
            t_ref[...], w_ref[0],
            (((2,), (0,)), ((), ())),
            preferred_element_type=jnp.float32)

    out = pl.pallas_call(
        body,
        grid=(r // rows_per_block,),
        in_specs=[
            pl.BlockSpec((rows_per_block, s, e), lambda i: (i, 0, 0)),
            pl.BlockSpec((1, e), lambda i: (0, 0)),
        ],
        out_specs=pl.BlockSpec((rows_per_block, s), lambda i: (i, 0)),
        out_shape=jax.ShapeDtypeStruct((r, s), jnp.float32),
    )(t3, w.reshape(1, e))
    return out.reshape(n)


def _score_gather(ts, seqs):
    b, l = seqs.shape
    per_w = (b * l) // NW
    c = per_w // 128
    k = 4
    seq3 = seqs.reshape(NW, c, 128)
    mesh = plsc.VectorSubcoreMesh(core_axis_name="c", subcore_axis_name="s")

    @functools.partial(
        pl.kernel,
        out_type=jax.ShapeDtypeStruct((NW, c, 128), jnp.float32),
        mesh=mesh,
        scratch_types=[
            pltpu.VMEM((c, 128), jnp.int32),
            pltpu.VMEM((c, 128), jnp.float32),
            pltpu.SemaphoreType.DMA,
        ],
    )
    def kern(ts_hbm, seq_hbm, out_hbm, idx_v, val_v, sem):
        wid = lax.axis_index("s") * NC + lax.axis_index("c")
        pltpu.sync_copy(seq_hbm.at[wid], idx_v)

        def group(g, carry):
            for jj in range(k):
                j = g * k + jj
                pltpu.async_copy(ts_hbm.at[idx_v.at[j]], val_v.at[j], sem)
            for jj in range(k):
                j = g * k + jj
                pltpu.make_async_copy(
                    ts_hbm.at[idx_v.at[j]], val_v.at[j], sem).wait()
            return carry

        lax.fori_loop(0, c // k, group, 0)
        pltpu.sync_copy(val_v, out_hbm.at[wid])

    return kern(ts, seq3).reshape(b, l)


def _attn_weights(scores, seqs, gamma, beta):
    b, l = scores.shape

    def body(s_ref, q_ref, g_ref, bt_ref, o_ref):
        s = s_ref[...]
        mean = jnp.mean(s, axis=0, keepdims=True)
        var = jnp.mean((s - mean) ** 2, axis=0, keepdims=True)
        xn = g_ref[...] * (s - mean) / jnp.sqrt(var + BN_EPS) + bt_ref[...]
        mask = (q_ref[...] != 0).astype(jnp.float32)
        o_ref[...] = jax.nn.sigmoid(xn) * mask

    return pl.pallas_call(
        body,
        out_shape=jax.ShapeDtypeStruct((b, l), jnp.float32),
    )(scores, seqs, gamma.reshape(1, l), beta.reshape(1, l))


def _pool(table, seqs, weights):
    b, l = seqs.shape
    n, e = table.shape
    bpw = b // NW
    lp = -(-l // 16) * 16
    if lp != l:
        seqs = jnp.pad(seqs, ((0, 0), (0, lp - l)))
        weights = jnp.pad(weights, ((0, 0), (0, lp - l)))
    cs = lp if lp <= 128 else lp // 2
    ch = lp // cs
    seq4 = seqs.reshape(NW, bpw, ch, cs)
    w3 = weights.reshape(NW, bpw, lp)
    mesh = plsc.VectorSubcoreMesh(core_axis_name="c", subcore_axis_name="s")

    @functools.partial(
        pl.kernel,
        out_type=jax.ShapeDtypeStruct((NW, bpw, e), jnp.float32),
        mesh=mesh,
        scratch_types=[
            pltpu.VMEM((bpw, ch, cs), jnp.int32),
            pltpu.VMEM((bpw, lp), jnp.float32),
            pltpu.VMEM((2, lp, e), jnp.float32),
            pltpu.VMEM((bpw, e), jnp.float32),
            pltpu.SemaphoreType.DMA((2,)),
        ],
        compiler_params=pltpu.CompilerParams(use_tc_tiling_on_sc=False),
    )
    def kern(tab_hbm, seq_hbm, w_hbm, out_hbm, idx_v, wgt_v, rows_v, out_v, sems):
        wid = lax.axis_index("s") * NC + lax.axis_index("c")
        pltpu.sync_copy(seq_hbm.at[wid], idx_v)
        pltpu.sync_copy(w_hbm.at[wid], wgt_v)

        def fire(bi, slot):
            for j in range(ch):
                pltpu.async_copy(
                    tab_hbm.at[idx_v.at[bi, j]],
                    rows_v.at[slot, pl.ds(j * cs, cs)],
                    sems.at[slot])

        def wait(bi, slot):
            for j in range(ch):
                pltpu.make_async_copy(
                    tab_hbm.at[idx_v.at[bi, j]],
                    rows_v.at[slot, pl.ds(j * cs, cs)],
                    sems.at[slot]).wait()

        fire(0, 0)

        def body(bi, carry):
            slot = lax.rem(bi, 2)

            @pl.when(bi + 1 < bpw)
            def _():
                fire(bi + 1, lax.rem(bi + 1, 2))

            wait(bi, slot)

            zero = jnp.zeros((16,), jnp.float32)

            def inner(lg, acc):
                a0, a1, a2, a3, wsv = acc
                wvec = wgt_v[bi, pl.ds(lg * 16, 16)]
                for kk in range(16):
                    wv = wvec[kk]
                    li = lg * 16 + kk
                    a0 = a0 + wv * rows_v[slot, li, pl.ds(0, 16)]
                    a1 = a1 + wv * rows_v[slot, li, pl.ds(16, 16)]
                    a2 = a2 + wv * rows_v[slot, li, pl.ds(32, 16)]
                    a3 = a3 + wv * rows_v[slot, li, pl.ds(48, 16)]
                return (a0, a1, a2, a3, wsv + wvec)

            a0, a1, a2, a3, wsv = lax.fori_loop(
                0, lp // 16, inner, (zero, zero, zero, zero, zero))
            ws = wsv[0]
            for kk in range(1, 16):
                ws = ws + wsv[kk]
            d = ws + EPS
            out_v[bi, pl.ds(0, 16)] = a0 / d
            out_v[bi, pl.ds(16, 16)] = a1 / d
            out_v[bi, pl.ds(32, 16)] = a2 / d
            out_v[bi, pl.ds(48, 16)] = a3 / d
            return carry

        lax.fori_loop(0, bpw, body, 0)
        pltpu.sync_copy(out_v, out_hbm.at[wid])

    return kern(table, seq4, w3).reshape(b, e)


def _similarity(pq, pc):
    b, e = pq.shape
    tb = 256

    def body(q_ref, c_ref, o_ref):
        q = q_ref[...]
        c = c_ref[...]
        qn = q / (jnp.sqrt(jnp.sum(q * q, axis=1, keepdims=True)) + EPS)
        cn = c / (jnp.sqrt(jnp.sum(c * c, axis=1, keepdims=True)) + EPS)
        o_ref[...] = lax.dot_general(
            qn, cn, (((1,), (1,)), ((), ())),
            preferred_element_type=jnp.float32)

    return pl.pallas_call(
        body,
        grid=(b // tb, b // tb),
        in_specs=[
            pl.BlockSpec((tb, e), lambda i, j: (i, 0)),
            pl.BlockSpec((tb, e), lambda i, j: (j, 0)),
        ],
        out_specs=pl.BlockSpec((tb, tb), lambda i, j: (i, j)),
        out_shape=jax.ShapeDtypeStruct((b, b), jnp.float32),
    )(pq, pc)


def _encode(seqs, table, w, gamma, beta, rows_per_block):
    ts = _row_scores(table, w, rows_per_block)
    scores = _score_gather(ts, seqs)
    weights = _attn_weights(scores, seqs, gamma, beta)
    return _pool(table, seqs, weights)


def kernel(code_seqs, query_seqs, code_table, code_w, code_gamma, code_beta,
           query_table, query_w, query_gamma, query_beta):
    pq = _encode(query_seqs, query_table, query_w, query_gamma, query_beta, 80)
    pc = _encode(code_seqs, code_table, code_w, code_gamma, code_beta, 80)
    return _similarity(pq, pc)


# speedup vs baseline: 1.4953x; 1.4953x over previous
import functools

import jax
import jax.numpy as jnp
from jax import lax
from jax.experimental import pallas as pl
from jax.experimental.pallas import tpu as pltpu
from jax.experimental.pallas import tpu_sc as plsc

NC, NS = 2, 16
NW = NC * NS
EPS = 1e-8
BN_EPS = 1e-5


def _row_scores(table, w, rows_per_block):
    n, e = table.shape
    grid = n // rows_per_block
    sub = rows_per_block // 8

    def body(t_ref, w_ref, o_ref):
        t = t_ref[...].reshape(sub, 8, e)
        o_ref[...] = lax.dot_general(
            t, w_ref[0],
            (((2,), (0,)), ((), ())),
            preferred_element_type=jnp.float32)

    out = pl.pallas_call(
        body,
        grid=(grid,),
        in_specs=[
            pl.BlockSpec((rows_per_block, e), lambda i: (i, 0)),
            pl.BlockSpec((1, e), lambda i: (0, 0)),
        ],
        out_specs=pl.BlockSpec((sub, 8), lambda i: (i, 0)),
        out_shape=jax.ShapeDtypeStruct((n // 8, 8), jnp.float32),
    )(table, w.reshape(1, e))
    return out.reshape(n)


def _row_scores_3d(table, w, rows_per_block):
    n, e = table.shape
    s = 125
    r = n // s
    t3 = table.reshape(r, s, e)

    def body(t_ref, w_ref, o_ref):
        o_ref[...] = lax.dot_general(
            t_ref[...], w_ref[0],
            (((2,), (0,)), ((), ())),
            preferred_element_type=jnp.float32)

    out = pl.pallas_call(
        body,
        grid=(r // rows_per_block,),
        in_specs=[
            pl.BlockSpec((rows_per_block, s, e), lambda i: (i, 0, 0)),
            pl.BlockSpec((1, e), lambda i: (0, 0)),
        ],
        out_specs=pl.BlockSpec((rows_per_block, s), lambda i: (i, 0)),
        out_shape=jax.ShapeDtypeStruct((r, s), jnp.float32),
    )(t3, w.reshape(1, e))
    return out.reshape(n)


def _score_gather(ts, seq_flat):
    t = seq_flat.shape[0]
    per_w = t // NW
    seq2 = seq_flat.reshape(NW, per_w)
    mesh = plsc.VectorSubcoreMesh(core_axis_name="c", subcore_axis_name="s")

    @functools.partial(
        pl.kernel,
        out_type=jax.ShapeDtypeStruct((NW, per_w), jnp.float32),
        mesh=mesh,
        scratch_types=[
            pltpu.VMEM((per_w,), jnp.int32),
            pltpu.VMEM((per_w,), jnp.float32),
            pltpu.SemaphoreType.DMA,
        ],
    )
    def kern(ts_hbm, seq_hbm, out_hbm, idx_v, val_v, sem):
        wid = lax.axis_index("s") * NC + lax.axis_index("c")
        pltpu.sync_copy(seq_hbm.at[wid], idx_v)
        pltpu.async_copy(ts_hbm.at[idx_v], val_v, sem).wait()
        pltpu.sync_copy(val_v, out_hbm.at[wid])

    return kern(ts, seq2).reshape(t)


def _attn_weights_t(scores_t, seqs_t, gamma, beta, lb):
    l, b = scores_t.shape

    def body(s_ref, q_ref, g_ref, bt_ref, o_ref):
        s = s_ref[...]
        mean = jnp.mean(s, axis=1, keepdims=True)
        var = jnp.mean((s - mean) ** 2, axis=1, keepdims=True)
        xn = g_ref[...] * (s - mean) / jnp.sqrt(var + BN_EPS) + bt_ref[...]
        mask = (q_ref[...] != 0).astype(jnp.float32)
        o_ref[...] = jax.nn.sigmoid(xn) * mask

    return pl.pallas_call(
        body,
        grid=(l // lb,),
        in_specs=[
            pl.BlockSpec((lb, b), lambda i: (i, 0)),
            pl.BlockSpec((lb, b), lambda i: (i, 0)),
            pl.BlockSpec((lb, 1), lambda i: (i, 0)),
            pl.BlockSpec((lb, 1), lambda i: (i, 0)),
        ],
        out_specs=pl.BlockSpec((lb, b), lambda i: (i, 0)),
        out_shape=jax.ShapeDtypeStruct((l, b), jnp.float32),
    )(scores_t, seqs_t, gamma.reshape(l, 1), beta.reshape(l, 1))


def _pool(table, seqs, weights):
    b, l = seqs.shape
    n, e = table.shape
    bpw = b // NW
    lp = -(-l // 16) * 16
    if lp != l:
        seqs = jnp.pad(seqs, ((0, 0), (0, lp - l)))
        weights = jnp.pad(weights, ((0, 0), (0, lp - l)))
    nbuf = 4
    seq3 = seqs.reshape(NW, bpw, lp)
    w3 = weights.reshape(NW, bpw, lp)
    mesh = plsc.VectorSubcoreMesh(core_axis_name="c", subcore_axis_name="s")

    @functools.partial(
        pl.kernel,
        out_type=jax.ShapeDtypeStruct((NW, bpw, e), jnp.float32),
        mesh=mesh,
        scratch_types=[
            pltpu.VMEM((bpw, lp), jnp.int32),
            pltpu.VMEM((bpw, lp), jnp.float32),
            pltpu.VMEM((nbuf, lp, e), jnp.float32),
            pltpu.VMEM((bpw, e), jnp.float32),
            pltpu.SemaphoreType.DMA((nbuf,)),
        ],
        compiler_params=pltpu.CompilerParams(use_tc_tiling_on_sc=False),
    )
    def kern(tab_hbm, seq_hbm, w_hbm, out_hbm, idx_v, wgt_v, rows_v, out_v, sems):
        wid = lax.axis_index("s") * NC + lax.axis_index("c")
        pltpu.sync_copy(seq_hbm.at[wid], idx_v)
        pltpu.sync_copy(w_hbm.at[wid], wgt_v)

        def fire(bi, slot):
            pltpu.async_copy(
                tab_hbm.at[idx_v.at[bi]], rows_v.at[slot], sems.at[slot])

        def wait(bi, slot):
            pltpu.make_async_copy(
                tab_hbm.at[idx_v.at[bi]], rows_v.at[slot], sems.at[slot]).wait()

        for p in range(nbuf - 1):
            fire(p, p)

        def body(bi, carry):
            slot = lax.rem(bi, nbuf)

            @pl.when(bi + nbuf - 1 < bpw)
            def _():
                fire(bi + nbuf - 1, lax.rem(bi + nbuf - 1, nbuf))

            wait(bi, slot)

            zero = jnp.zeros((16,), jnp.float32)

            def inner(lg, acc):
                a0, a1, a2, a3, wsv = acc
                wvec = wgt_v[bi, pl.ds(lg * 16, 16)]
                for kk in range(16):
                    wv = wvec[kk]
                    li = lg * 16 + kk
                    a0 = a0 + wv * rows_v[slot, li, pl.ds(0, 16)]
                    a1 = a1 + wv * rows_v[slot, li, pl.ds(16, 16)]
                    a2 = a2 + wv * rows_v[slot, li, pl.ds(32, 16)]
                    a3 = a3 + wv * rows_v[slot, li, pl.ds(48, 16)]
                return (a0, a1, a2, a3, wsv + wvec)

            a0, a1, a2, a3, wsv = lax.fori_loop(
                0, lp // 16, inner, (zero, zero, zero, zero, zero))
            ws = wsv[0]
            for kk in range(1, 16):
                ws = ws + wsv[kk]
            d = ws + EPS
            out_v[bi, pl.ds(0, 16)] = a0 / d
            out_v[bi, pl.ds(16, 16)] = a1 / d
            out_v[bi, pl.ds(32, 16)] = a2 / d
            out_v[bi, pl.ds(48, 16)] = a3 / d
            return carry

        lax.fori_loop(0, bpw, body, 0)
        pltpu.sync_copy(out_v, out_hbm.at[wid])

    return kern(table, seq3, w3).reshape(b, e)


def _similarity(pq, pc):
    b, e = pq.shape
    tb = 256

    def body(q_ref, c_ref, o_ref):
        q = q_ref[...]
        c = c_ref[...]
        qn = q / (jnp.sqrt(jnp.sum(q * q, axis=1, keepdims=True)) + EPS)
        cn = c / (jnp.sqrt(jnp.sum(c * c, axis=1, keepdims=True)) + EPS)
        o_ref[...] = lax.dot_general(
            qn, cn, (((1,), (1,)), ((), ())),
            preferred_element_type=jnp.float32)

    return pl.pallas_call(
        body,
        grid=(b // tb, b // tb),
        in_specs=[
            pl.BlockSpec((tb, e), lambda i, j: (i, 0)),
            pl.BlockSpec((tb, e), lambda i, j: (j, 0)),
        ],
        out_specs=pl.BlockSpec((tb, tb), lambda i, j: (i, j)),
        out_shape=jax.ShapeDtypeStruct((b, b), jnp.float32),
    )(pq, pc)


def _encode(seqs, table, w, gamma, beta, rows_per_block, lb):
    b, l = seqs.shape
    if (table.shape[0] % rows_per_block == 0
            and (rows_per_block // 8) % 8 == 0):
        ts = _row_scores(table, w, rows_per_block)
    else:
        ts = _row_scores_3d(table, w, rows_per_block)
    seqs_t = seqs.T
    scores_t = _score_gather(ts, seqs_t.reshape(-1)).reshape(l, b)
    weights_t = _attn_weights_t(scores_t, seqs_t, gamma, beta, lb)
    return _pool(table, seqs, weights_t.T)


def kernel(code_seqs, query_seqs, code_table, code_w, code_gamma, code_beta,
           query_table, query_w, query_gamma, query_beta):
    pq = _encode(query_seqs, query_table, query_w, query_gamma, query_beta,
                 80, 20)
    pc = _encode(code_seqs, code_table, code_w, code_gamma, code_beta,
                 8000, 40)
    return _similarity(pq, pc)


# speedup vs baseline: 1.5431x; 1.0320x over previous
import functools

import jax
import jax.numpy as jnp
from jax import lax
from jax.experimental import pallas as pl
from jax.experimental.pallas import tpu as pltpu
from jax.experimental.pallas import tpu_sc as plsc

NC, NS = 2, 16
NW = NC * NS
EPS = 1e-8
BN_EPS = 1e-5


def _row_scores(table, w, rows_per_block):
    n, e = table.shape
    grid = n // rows_per_block
    sub = rows_per_block // 8

    def body(t_ref, w_ref, o_ref):
        t = t_ref[...].reshape(sub, 8, e)
        o_ref[...] = lax.dot_general(
            t, w_ref[0],
            (((2,), (0,)), ((), ())),
            preferred_element_type=jnp.float32)

    out = pl.pallas_call(
        body,
        grid=(grid,),
        in_specs=[
            pl.BlockSpec((rows_per_block, e), lambda i: (i, 0)),
            pl.BlockSpec((1, e), lambda i: (0, 0)),
        ],
        out_specs=pl.BlockSpec((sub, 8), lambda i: (i, 0)),
        out_shape=jax.ShapeDtypeStruct((n // 8, 8), jnp.float32),
    )(table, w.reshape(1, e))
    return out.reshape(n)


def _row_scores_3d(table, w, rows_per_block):
    n, e = table.shape
    s = 125
    r = n // s
    t3 = table.reshape(r, s, e)

    def body(t_ref, w_ref, o_ref):
        o_ref[...] = lax.dot_general(
            t_ref[...], w_ref[0],
            (((2,), (0,)), ((), ())),
            preferred_element_type=jnp.float32)

    out = pl.pallas_call(
        body,
        grid=(r // rows_per_block,),
        in_specs=[
            pl.BlockSpec((rows_per_block, s, e), lambda i: (i, 0, 0)),
            pl.BlockSpec((1, e), lambda i: (0, 0)),
        ],
        out_specs=pl.BlockSpec((rows_per_block, s), lambda i: (i, 0)),
        out_shape=jax.ShapeDtypeStruct((r, s), jnp.float32),
    )(t3, w.reshape(1, e))
    return out.reshape(n)


def _score_gather(ts, seqs):
    b, l = seqs.shape
    t = b * l
    n = ts.shape[0]
    per_w = t // NW
    seq2 = seqs.reshape(NW, per_w)
    chunk = 10000
    nch = n // chunk
    assert nch * chunk == n
    mesh = plsc.VectorSubcoreMesh(core_axis_name="c", subcore_axis_name="s")

    @functools.partial(
        pl.kernel,
        out_type=jax.ShapeDtypeStruct((NW, per_w), jnp.float32),
        mesh=mesh,
        scratch_types=[
            pltpu.VMEM((per_w,), jnp.int32),
            pltpu.VMEM((per_w,), jnp.float32),
            pltpu.VMEM((chunk,), jnp.float32),
            pltpu.VMEM_SHARED((n,), jnp.float32),
            pltpu.SemaphoreType.DMA,
        ],
    )
    def kern(ts_hbm, seq_hbm, out_hbm, idx_v, val_v, stg_v, ts_spm, sem):
        sid = lax.axis_index("s")
        wid = sid * NC + lax.axis_index("c")

        def fill(k, carry):
            @pl.when(lax.rem(k, NS) == sid)
            def _():
                off = pl.multiple_of(k * chunk, 8)
                pltpu.sync_copy(ts_hbm.at[pl.ds(off, chunk)], stg_v)
                pltpu.sync_copy(stg_v, ts_spm.at[pl.ds(off, chunk)])
            return carry

        lax.fori_loop(0, nch, fill, 0)
        pltpu.sync_copy(seq_hbm.at[wid], idx_v)
        plsc.subcore_barrier()
        pltpu.async_copy(ts_spm.at[idx_v], val_v, sem).wait()
        pltpu.sync_copy(val_v, out_hbm.at[wid])

    return kern(ts, seq2).reshape(b, l)


def _bn_partials(scores3):
    g, bb, l = scores3.shape

    def body(s_ref, s1_ref, s2_ref):
        s = s_ref[0]
        s1_ref[...] = jnp.sum(s, axis=0).reshape(1, 1, l)
        s2_ref[...] = jnp.sum(s * s, axis=0).reshape(1, 1, l)

    return pl.pallas_call(
        body,
        grid=(g,),
        in_specs=[pl.BlockSpec((1, bb, l), lambda i: (i, 0, 0))],
        out_specs=[
            pl.BlockSpec((1, 1, l), lambda i: (i, 0, 0)),
            pl.BlockSpec((1, 1, l), lambda i: (i, 0, 0)),
        ],
        out_shape=[
            jax.ShapeDtypeStruct((g, 1, l), jnp.float32),
            jax.ShapeDtypeStruct((g, 1, l), jnp.float32),
        ],
    )(scores3)


def _weights_apply(scores3, seqs3, s1, s2, gamma, beta, batch):
    g, bb, l = scores3.shape

    def body(s_ref, q_ref, s1_ref, s2_ref, g_ref, bt_ref, o_ref):
        mean = jnp.sum(s1_ref[...], axis=0) / batch
        msq = jnp.sum(s2_ref[...], axis=0) / batch
        var = msq - mean * mean
        s = s_ref[0]
        xn = g_ref[...] * (s - mean) / jnp.sqrt(var + BN_EPS) + bt_ref[...]
        mask = (q_ref[0] != 0).astype(jnp.float32)
        o_ref[...] = (jax.nn.sigmoid(xn) * mask).reshape(1, bb, l)

    return pl.pallas_call(
        body,
        grid=(g,),
        in_specs=[
            pl.BlockSpec((1, bb, l), lambda i: (i, 0, 0)),
            pl.BlockSpec((1, bb, l), lambda i: (i, 0, 0)),
            pl.BlockSpec((g, 1, l), lambda i: (0, 0, 0)),
            pl.BlockSpec((g, 1, l), lambda i: (0, 0, 0)),
            pl.BlockSpec((1, l), lambda i: (0, 0)),
            pl.BlockSpec((1, l), lambda i: (0, 0)),
        ],
        out_specs=pl.BlockSpec((1, bb, l), lambda i: (i, 0, 0)),
        out_shape=jax.ShapeDtypeStruct((g, bb, l), jnp.float32),
    )(scores3, seqs3, s1, s2, gamma.reshape(1, l), beta.reshape(1, l))


def _pool(table, seqs, weights):
    b, l = seqs.shape
    n, e = table.shape
    bpw = b // NW
    lp = -(-l // 16) * 16
    if lp != l:
        seqs = jnp.pad(seqs, ((0, 0), (0, lp - l)))
        weights = jnp.pad(weights, ((0, 0), (0, lp - l)))
    nbuf = 4
    seq3 = seqs.reshape(NW, bpw, lp)
    w3 = weights.reshape(NW, bpw, lp)
    mesh = plsc.VectorSubcoreMesh(core_axis_name="c", subcore_axis_name="s")

    @functools.partial(
        pl.kernel,
        out_type=jax.ShapeDtypeStruct((NW, bpw, e), jnp.float32),
        mesh=mesh,
        scratch_types=[
            pltpu.VMEM((bpw, lp), jnp.int32),
            pltpu.VMEM((bpw, lp), jnp.float32),
            pltpu.VMEM((nbuf, lp, e), jnp.float32),
            pltpu.VMEM((bpw, e), jnp.float32),
            pltpu.SemaphoreType.DMA((nbuf,)),
        ],
        compiler_params=pltpu.CompilerParams(use_tc_tiling_on_sc=False),
    )
    def kern(tab_hbm, seq_hbm, w_hbm, out_hbm, idx_v, wgt_v, rows_v, out_v, sems):
        wid = lax.axis_index("s") * NC + lax.axis_index("c")
        pltpu.sync_copy(seq_hbm.at[wid], idx_v)
        pltpu.sync_copy(w_hbm.at[wid], wgt_v)

        def fire(bi, slot):
            pltpu.async_copy(
                tab_hbm.at[idx_v.at[bi]], rows_v.at[slot], sems.at[slot])

        def wait(bi, slot):
            pltpu.make_async_copy(
                tab_hbm.at[idx_v.at[bi]], rows_v.at[slot], sems.at[slot]).wait()

        for p in range(nbuf - 1):
            fire(p, p)

        def body(bi, carry):
            slot = lax.rem(bi, nbuf)

            @pl.when(bi + nbuf - 1 < bpw)
            def _():
                fire(bi + nbuf - 1, lax.rem(bi + nbuf - 1, nbuf))

            wait(bi, slot)

            zero = jnp.zeros((16,), jnp.float32)

            def inner(lg, acc):
                a0, a1, a2, a3, wsv = acc
                wvec = wgt_v[bi, pl.ds(lg * 16, 16)]
                for kk in range(16):
                    wv = wvec[kk]
                    li = lg * 16 + kk
                    a0 = a0 + wv * rows_v[slot, li, pl.ds(0, 16)]
                    a1 = a1 + wv * rows_v[slot, li, pl.ds(16, 16)]
                    a2 = a2 + wv * rows_v[slot, li, pl.ds(32, 16)]
                    a3 = a3 + wv * rows_v[slot, li, pl.ds(48, 16)]
                return (a0, a1, a2, a3, wsv + wvec)

            a0, a1, a2, a3, wsv = lax.fori_loop(
                0, lp // 16, inner, (zero, zero, zero, zero, zero))
            ws = wsv[0]
            for kk in range(1, 16):
                ws = ws + wsv[kk]
            d = ws + EPS
            out_v[bi, pl.ds(0, 16)] = a0 / d
            out_v[bi, pl.ds(16, 16)] = a1 / d
            out_v[bi, pl.ds(32, 16)] = a2 / d
            out_v[bi, pl.ds(48, 16)] = a3 / d
            return carry

        lax.fori_loop(0, bpw, body, 0)
        pltpu.sync_copy(out_v, out_hbm.at[wid])

    return kern(table, seq3, w3).reshape(b, e)


def _similarity(pq, pc):
    b, e = pq.shape
    ti, tj = 256, 2048

    def body(q_ref, c_ref, o_ref):
        q = q_ref[...]
        c = c_ref[...]
        qn = q / (jnp.sqrt(jnp.sum(q * q, axis=1, keepdims=True)) + EPS)
        cn = c / (jnp.sqrt(jnp.sum(c * c, axis=1, keepdims=True)) + EPS)
        o_ref[...] = lax.dot_general(
            qn, cn, (((1,), (1,)), ((), ())),
            preferred_element_type=jnp.float32)

    return pl.pallas_call(
        body,
        grid=(b // ti, b // tj),
        in_specs=[
            pl.BlockSpec((ti, e), lambda i, j: (i, 0)),
            pl.BlockSpec((tj, e), lambda i, j: (j, 0)),
        ],
        out_specs=pl.BlockSpec((ti, tj), lambda i, j: (i, j)),
        out_shape=jax.ShapeDtypeStruct((b, b), jnp.float32),
    )(pq, pc)


def _encode(seqs, table, w, gamma, beta, rows_per_block):
    b, l = seqs.shape
    if (table.shape[0] % rows_per_block == 0
            and (rows_per_block // 8) % 8 == 0):
        ts = _row_scores(table, w, rows_per_block)
    else:
        ts = _row_scores_3d(table, w, rows_per_block)
    scores = _score_gather(ts, seqs)
    scores3 = scores.reshape(NW, b // NW, l)
    seqs3 = seqs.reshape(NW, b // NW, l)
    s1, s2 = _bn_partials(scores3)
    weights = _weights_apply(scores3, seqs3, s1, s2, gamma, beta,
                             float(b)).reshape(b, l)
    return _pool(table, seqs, weights)


def kernel(code_seqs, query_seqs, code_table, code_w, code_gamma, code_beta,
           query_table, query_w, query_gamma, query_beta):
    pq = _encode(query_seqs, query_table, query_w, query_gamma, query_beta, 80)
    pc = _encode(code_seqs, code_table, code_w, code_gamma, code_beta, 8000)
    return _similarity(pq, pc)
